# baseline (speedup 1.0000x reference)
import functools
import math

import jax
import jax.numpy as jnp
from jax.experimental import pallas as pl
from jax.experimental.pallas import tpu as pltpu

_T = 32768
_D = 1024
_H = 1024
_K = 256
_TB = 2048
_NBLK = _T // _TB


def _tc_body(vs_ref, v_ref, wih_ref, whh_ref, x_ref, h_ref, wv_ref,
             bih_ref, bhh_ref, alpha_ref, hnew_ref, sv_ref):
    i = pl.program_id(0)
    blk = vs_ref[...]
    alpha_ref[...] = jax.lax.dot_general(
        v_ref[...], blk, (((1,), (1,)), ((), ())),
        preferred_element_type=jnp.float32).reshape(1, 1, _TB)

    @pl.when(i == _NBLK - 1)
    def _gru():
        gi = jnp.dot(wih_ref[...], x_ref[...],
                     preferred_element_type=jnp.float32) + bih_ref[...]
        gh = jnp.dot(whh_ref[...], h_ref[...],
                     preferred_element_type=jnp.float32) + bhh_ref[...]
        i_r, i_z, i_n = gi[:_H], gi[_H:2 * _H], gi[2 * _H:]
        h_r, h_z, h_n = gh[:_H], gh[_H:2 * _H], gh[2 * _H:]
        r = jax.nn.sigmoid(i_r + h_r)
        z = jax.nn.sigmoid(i_z + h_z)
        n = jnp.tanh(i_n + r * h_n)
        hnew_ref[...] = (1.0 - z) * n + z * h_ref[...]
        sv_ref[...] = jnp.dot(wv_ref[...], v_ref[...].reshape(_D, 1),
                              preferred_element_type=jnp.float32)


@functools.partial(jax.jit, static_argnames=())
def _tc_part(vs, v_row, w_ih, w_hh, x_col, h_col, wv_row, b_ih_col, b_hh_col):
    return pl.pallas_call(
        _tc_body,
        grid=(_NBLK,),
        in_specs=[
            pl.BlockSpec((_TB, _D), lambda i: (i, 0)),
            pl.BlockSpec((1, _D), lambda i: (0, 0)),
            pl.BlockSpec((3 * _H, _D + 1), lambda i: (0, 0)),
            pl.BlockSpec((3 * _H, _H), lambda i: (0, 0)),
            pl.BlockSpec((_D + 1, 1), lambda i: (0, 0)),
            pl.BlockSpec((_H, 1), lambda i: (0, 0)),
            pl.BlockSpec((1, _D), lambda i: (0, 0)),
            pl.BlockSpec((3 * _H, 1), lambda i: (0, 0)),
            pl.BlockSpec((3 * _H, 1), lambda i: (0, 0)),
        ],
        out_specs=[
            pl.BlockSpec((1, 1, _TB), lambda i: (i, 0, 0)),
            pl.BlockSpec((_H, 1), lambda i: (0, 0)),
            pl.BlockSpec((1, 1), lambda i: (0, 0)),
        ],
        out_shape=[
            jax.ShapeDtypeStruct((_NBLK, 1, _TB), jnp.float32),
            jax.ShapeDtypeStruct((_H, 1), jnp.float32),
            jax.ShapeDtypeStruct((1, 1), jnp.float32),
        ],
    )(vs, v_row, w_ih, w_hh, x_col, h_col, wv_row, b_ih_col, b_hh_col)


def kernel(v, s, t, vs, hs, ts, W_score, b_score, W_ih, W_hh, b_ih, b_hh):
    v_row = v.reshape(1, _D)
    x_col = jnp.concatenate([v, s]).reshape(_D + 1, 1)
    h_col = hs[-1, 0].reshape(_H, 1)
    wv_row = W_score[:, :_D]

    alpha_blk, hnew_col, sv = _tc_part(
        vs, v_row, W_ih, W_hh, x_col, h_col, wv_row,
        b_ih.reshape(3 * _H, 1), b_hh.reshape(3 * _H, 1))
    alpha = alpha_blk.reshape(_T)

    ts_d = t - ts
    alpha_top, idx = jax.lax.top_k(alpha, _K)
    decay = (1.0 - 1e-07) ** ts_d[idx]
    alpha_top = alpha_top * decay
    alpha_sm = jax.nn.softmax(alpha_top.reshape(1, -1), axis=-1)
    hs2 = hs.reshape(-1, _H)
    attn_h = (alpha_sm @ hs2[idx]).reshape(-1)
    score_attn = attn_h @ W_score[0, _D:]

    score = (sv[0, 0] + score_attn + b_score[0]).reshape(1, 1)
    h_new = hnew_col.reshape(1, 1, _H)
    return (score, h_new)
